# Initial kernel scaffold; baseline (speedup 1.0000x reference)
import jax
import jax.numpy as jnp
from jax.experimental import pallas as pl


def kernel(obs, pdf_grid):
    raise NotImplementedError("write your pallas kernel here")


# speedup vs baseline: 31.6028x; 31.6028x over previous
import functools

import jax
import jax.numpy as jnp
import numpy as np
from jax import lax
from jax.experimental import pallas as pl
from jax.experimental.pallas import tpu as pltpu
from jax.experimental.pallas import tpu_sc as plsc

_N = 1048576
_G = 128
_NC = 2
_NS = 16
_NW = _NC * _NS
_B = _N // _NW
_L = 16
_EPS = np.float32(1e-10)
_STEP = np.float32(1.0 / 127.0)
_TARGET_I = np.int32(_G - 1)


def _tec_body(obs_hbm, grid_hbm, out_hbm, obs_v, grid_v, out_v):
    wid = lax.axis_index("s") * _NC + lax.axis_index("c")
    base = wid * _B
    pltpu.sync_copy(obs_hbm.at[pl.ds(2 * base, 2 * _B)], obs_v)
    pltpu.sync_copy(grid_hbm, grid_v)

    lanes2 = lax.iota(jnp.int32, _L) * 2

    def body(i, carry):
        o = i * _L
        iu = lanes2 + (2 * o)
        u = plsc.load_gather(obs_v, [iu])
        v = plsc.load_gather(obs_v, [iu + 1])

        fu = jnp.minimum(jnp.maximum(u, _EPS), np.float32(1.0) - _EPS) / _STEP
        fv = jnp.minimum(jnp.maximum(v, _EPS), np.float32(1.0) - _EPS) / _STEP
        i0u = fu.astype(jnp.int32)
        i0v = fv.astype(jnp.int32)
        du = fu - i0u.astype(jnp.float32)
        dv = fv - i0v.astype(jnp.float32)
        i1u = jnp.minimum(i0u + 1, _TARGET_I)
        i1v = jnp.minimum(i0v + 1, _TARGET_I)

        r0 = i0u << 7
        r1 = i1u << 7
        g00 = plsc.load_gather(grid_v, [r0 + i0v])
        g10 = plsc.load_gather(grid_v, [r1 + i0v])
        g01 = plsc.load_gather(grid_v, [r0 + i1v])
        g11 = plsc.load_gather(grid_v, [r1 + i1v])

        res = (g00
               + (g10 - g00) * du
               + (g01 - g00) * dv
               + (g11 - g01 - g10 + g00) * (du * dv))
        out_v[pl.ds(o, _L)] = jnp.maximum(res, np.float32(0.0))
        return carry

    lax.fori_loop(0, _B // _L, body, 0)
    pltpu.sync_copy(out_v, out_hbm.at[pl.ds(base, _B)])


def kernel(obs, pdf_grid):
    mesh = plsc.VectorSubcoreMesh(core_axis_name="c", subcore_axis_name="s")
    run = functools.partial(
        pl.kernel,
        mesh=mesh,
        out_type=jax.ShapeDtypeStruct((_N,), jnp.float32),
        compiler_params=pltpu.CompilerParams(needs_layout_passes=False),
        scratch_types=[
            pltpu.VMEM((2 * _B,), jnp.float32),
            pltpu.VMEM((_G * _G,), jnp.float32),
            pltpu.VMEM((_B,), jnp.float32),
        ],
    )(_tec_body)
    out = run(obs.reshape(-1), pdf_grid.reshape(-1))
    return out[:, None]


# speedup vs baseline: 32.1861x; 1.0185x over previous
import functools

import jax
import jax.numpy as jnp
import numpy as np
from jax import lax
from jax.experimental import pallas as pl
from jax.experimental.pallas import tpu as pltpu
from jax.experimental.pallas import tpu_sc as plsc

_N = 1048576
_G = 128
_NC = 2
_NS = 16
_NW = _NC * _NS
_B = _N // _NW
_L = 16
_EPS = np.float32(1e-10)
_STEP = np.float32(1.0 / 127.0)
_TARGET_I = np.int32(_G - 1)


def _tec_body(obs_hbm, grid_hbm, out_hbm, obs_v, grid_v, out_v):
    wid = lax.axis_index("s") * _NC + lax.axis_index("c")
    base = wid * _B
    pltpu.sync_copy(obs_hbm.at[pl.ds(2 * base, 2 * _B)], obs_v)
    pltpu.sync_copy(grid_hbm, grid_v)

    lanes2 = lax.iota(jnp.int32, _L) * 2

    @plsc.parallel_loop(0, _B, _L, unroll=8)
    def _loop(o):
        iu = lanes2 + (2 * o)
        u = plsc.load_gather(obs_v, [iu])
        v = plsc.load_gather(obs_v, [iu + 1])

        fu = jnp.minimum(jnp.maximum(u, _EPS), np.float32(1.0) - _EPS) / _STEP
        fv = jnp.minimum(jnp.maximum(v, _EPS), np.float32(1.0) - _EPS) / _STEP
        i0u = fu.astype(jnp.int32)
        i0v = fv.astype(jnp.int32)
        du = fu - i0u.astype(jnp.float32)
        dv = fv - i0v.astype(jnp.float32)
        i1u = jnp.minimum(i0u + 1, _TARGET_I)
        i1v = jnp.minimum(i0v + 1, _TARGET_I)

        r0 = i0u << 7
        r1 = i1u << 7
        g00 = plsc.load_gather(grid_v, [r0 + i0v])
        g10 = plsc.load_gather(grid_v, [r1 + i0v])
        g01 = plsc.load_gather(grid_v, [r0 + i1v])
        g11 = plsc.load_gather(grid_v, [r1 + i1v])

        res = (g00
               + (g10 - g00) * du
               + (g01 - g00) * dv
               + (g11 - g01 - g10 + g00) * (du * dv))
        out_v[pl.ds(o, _L)] = jnp.maximum(res, np.float32(0.0))

    pltpu.sync_copy(out_v, out_hbm.at[pl.ds(base, _B)])


def kernel(obs, pdf_grid):
    mesh = plsc.VectorSubcoreMesh(core_axis_name="c", subcore_axis_name="s")
    run = functools.partial(
        pl.kernel,
        mesh=mesh,
        out_type=jax.ShapeDtypeStruct((_N,), jnp.float32),
        compiler_params=pltpu.CompilerParams(needs_layout_passes=False),
        scratch_types=[
            pltpu.VMEM((2 * _B,), jnp.float32),
            pltpu.VMEM((_G * _G,), jnp.float32),
            pltpu.VMEM((_B,), jnp.float32),
        ],
    )(_tec_body)
    out = run(obs.reshape(-1), pdf_grid.reshape(-1))
    return out[:, None]


# speedup vs baseline: 745.3119x; 23.1563x over previous
import functools

import jax
import jax.numpy as jnp
import numpy as np
from jax import lax
from jax.experimental import pallas as pl
from jax.experimental.pallas import tpu as pltpu
from jax.experimental.pallas import tpu_sc as plsc

_N = 1048576
_G = 128
_NC = 2
_NS = 16
_NW = _NC * _NS
_B = _N // _NW
_L = 16
_EPS = np.float32(1e-10)
_STEP = np.float32(1.0 / 127.0)
_TARGET_I = np.int32(_G - 1)


def _tec_body(obs_hbm, grid_hbm, out_hbm, obs_v, grid_v, out_v):
    wid = lax.axis_index("s") * _NC + lax.axis_index("c")
    base = wid * _B
    pltpu.sync_copy(obs_hbm.at[pl.ds(base, _B)], obs_v.at[pl.ds(0, _B)])
    pltpu.sync_copy(obs_hbm.at[pl.ds(_N + base, _B)], obs_v.at[pl.ds(_B, _B)])
    pltpu.sync_copy(grid_hbm, grid_v)

    @plsc.parallel_loop(0, _B, _L, unroll=8)
    def _loop(o):
        u = obs_v[pl.ds(o, _L)]
        v = obs_v[pl.ds(_B + o, _L)]

        fu = jnp.minimum(jnp.maximum(u, _EPS), np.float32(1.0) - _EPS) / _STEP
        fv = jnp.minimum(jnp.maximum(v, _EPS), np.float32(1.0) - _EPS) / _STEP
        i0u = fu.astype(jnp.int32)
        i0v = fv.astype(jnp.int32)
        du = fu - i0u.astype(jnp.float32)
        dv = fv - i0v.astype(jnp.float32)
        i1u = jnp.minimum(i0u + 1, _TARGET_I)
        i1v = jnp.minimum(i0v + 1, _TARGET_I)

        r0 = i0u << 7
        r1 = i1u << 7
        g00 = plsc.load_gather(grid_v, [r0 + i0v])
        g10 = plsc.load_gather(grid_v, [r1 + i0v])
        g01 = plsc.load_gather(grid_v, [r0 + i1v])
        g11 = plsc.load_gather(grid_v, [r1 + i1v])

        res = (g00
               + (g10 - g00) * du
               + (g01 - g00) * dv
               + (g11 - g01 - g10 + g00) * (du * dv))
        out_v[pl.ds(o, _L)] = jnp.maximum(res, np.float32(0.0))

    pltpu.sync_copy(out_v, out_hbm.at[pl.ds(base, _B)])


def kernel(obs, pdf_grid):
    mesh = plsc.VectorSubcoreMesh(core_axis_name="c", subcore_axis_name="s")
    run = functools.partial(
        pl.kernel,
        mesh=mesh,
        out_type=jax.ShapeDtypeStruct((_N,), jnp.float32),
        compiler_params=pltpu.CompilerParams(needs_layout_passes=False),
        scratch_types=[
            pltpu.VMEM((2 * _B,), jnp.float32),
            pltpu.VMEM((_G * _G,), jnp.float32),
            pltpu.VMEM((_B,), jnp.float32),
        ],
    )(_tec_body)
    out = run(obs.T.reshape(-1), pdf_grid.reshape(-1))
    return out[:, None]


# speedup vs baseline: 757.5404x; 1.0164x over previous
import functools

import jax
import jax.numpy as jnp
import numpy as np
from jax import lax
from jax.experimental import pallas as pl
from jax.experimental.pallas import tpu as pltpu
from jax.experimental.pallas import tpu_sc as plsc

_N = 1048576
_G = 128
_NC = 2
_NS = 16
_NW = _NC * _NS
_B = _N // _NW
_L = 16
_EPS = np.float32(1e-10)
_STEP = np.float32(1.0 / 127.0)
_TARGET_I = np.int32(_G - 1)


def _tec_body(obs_hbm, grid_hbm, out_hbm, obs_v, grid_v, out_v, sem):
    wid = lax.axis_index("s") * _NC + lax.axis_index("c")
    base = wid * _B
    c0 = pltpu.async_copy(obs_hbm.at[pl.ds(base, _B)], obs_v.at[pl.ds(0, _B)], sem)
    c1 = pltpu.async_copy(obs_hbm.at[pl.ds(_N + base, _B)], obs_v.at[pl.ds(_B, _B)], sem)
    c2 = pltpu.async_copy(grid_hbm, grid_v, sem)
    c0.wait()
    c1.wait()
    c2.wait()

    @plsc.parallel_loop(0, _B, _L, unroll=8)
    def _loop(o):
        u = obs_v[pl.ds(o, _L)]
        v = obs_v[pl.ds(_B + o, _L)]

        fu = jnp.minimum(jnp.maximum(u, _EPS), np.float32(1.0) - _EPS) / _STEP
        fv = jnp.minimum(jnp.maximum(v, _EPS), np.float32(1.0) - _EPS) / _STEP
        i0u = fu.astype(jnp.int32)
        i0v = fv.astype(jnp.int32)
        du = fu - i0u.astype(jnp.float32)
        dv = fv - i0v.astype(jnp.float32)
        i1u = jnp.minimum(i0u + 1, _TARGET_I)
        i1v = jnp.minimum(i0v + 1, _TARGET_I)

        r0 = i0u << 7
        r1 = i1u << 7
        g00 = plsc.load_gather(grid_v, [r0 + i0v])
        g10 = plsc.load_gather(grid_v, [r1 + i0v])
        g01 = plsc.load_gather(grid_v, [r0 + i1v])
        g11 = plsc.load_gather(grid_v, [r1 + i1v])

        res = (g00
               + (g10 - g00) * du
               + (g01 - g00) * dv
               + (g11 - g01 - g10 + g00) * (du * dv))
        out_v[pl.ds(o, _L)] = jnp.maximum(res, np.float32(0.0))

    pltpu.sync_copy(out_v, out_hbm.at[0, pl.ds(base, _B)])


def kernel(obs, pdf_grid):
    mesh = plsc.VectorSubcoreMesh(core_axis_name="c", subcore_axis_name="s")
    run = functools.partial(
        pl.kernel,
        mesh=mesh,
        out_type=jax.ShapeDtypeStruct((1, _N), jnp.float32),
        compiler_params=pltpu.CompilerParams(
            needs_layout_passes=False, use_tc_tiling_on_sc=False),
        scratch_types=[
            pltpu.VMEM((2 * _B,), jnp.float32),
            pltpu.VMEM((_G * _G,), jnp.float32),
            pltpu.VMEM((_B,), jnp.float32),
            pltpu.SemaphoreType.DMA,
        ],
    )(_tec_body)
    return run(obs.T.reshape(-1), pdf_grid.reshape(-1)).reshape(_N, 1)


# speedup vs baseline: 899.6214x; 1.1876x over previous
import functools

import jax
import jax.numpy as jnp
import numpy as np
from jax import lax
from jax.experimental import pallas as pl
from jax.experimental.pallas import tpu as pltpu
from jax.experimental.pallas import tpu_sc as plsc

_N = 1048576
_G = 128
_NC = 2
_NS = 16
_NW = _NC * _NS
_B = _N // _NW
_L = 16
_EPS = np.float32(1e-10)
_STEP = np.float32(1.0 / 127.0)
_TARGET_I = np.int32(_G - 1)


def _tec_body(obs_hbm, grid_hbm, out_hbm, obs_v, grid_v, out_v, sem):
    wid = lax.axis_index("s") * _NC + lax.axis_index("c")
    base = wid * _B
    c0 = pltpu.async_copy(obs_hbm.at[pl.ds(2 * base, 2 * _B)], obs_v, sem)
    c1 = pltpu.async_copy(grid_hbm, grid_v, sem)
    c0.wait()
    c1.wait()

    @plsc.parallel_loop(0, _B, _L, unroll=8)
    def _loop(o):
        addr = o + ((o >> 7) << 7)
        u = obs_v[pl.ds(addr, _L)]
        v = obs_v[pl.ds(addr + 128, _L)]

        fu = jnp.minimum(jnp.maximum(u, _EPS), np.float32(1.0) - _EPS) / _STEP
        fv = jnp.minimum(jnp.maximum(v, _EPS), np.float32(1.0) - _EPS) / _STEP
        i0u = fu.astype(jnp.int32)
        i0v = fv.astype(jnp.int32)
        du = fu - i0u.astype(jnp.float32)
        dv = fv - i0v.astype(jnp.float32)
        i1u = jnp.minimum(i0u + 1, _TARGET_I)
        i1v = jnp.minimum(i0v + 1, _TARGET_I)

        r0 = i0u << 7
        r1 = i1u << 7
        g00 = plsc.load_gather(grid_v, [r0 + i0v])
        g10 = plsc.load_gather(grid_v, [r1 + i0v])
        g01 = plsc.load_gather(grid_v, [r0 + i1v])
        g11 = plsc.load_gather(grid_v, [r1 + i1v])

        res = (g00
               + (g10 - g00) * du
               + (g01 - g00) * dv
               + (g11 - g01 - g10 + g00) * (du * dv))
        out_v[pl.ds(o, _L)] = jnp.maximum(res, np.float32(0.0))

    pltpu.sync_copy(out_v, out_hbm.at[0, pl.ds(base, _B)])


def kernel(obs, pdf_grid):
    mesh = plsc.VectorSubcoreMesh(core_axis_name="c", subcore_axis_name="s")
    run = functools.partial(
        pl.kernel,
        mesh=mesh,
        out_type=jax.ShapeDtypeStruct((1, _N), jnp.float32),
        compiler_params=pltpu.CompilerParams(
            needs_layout_passes=False, use_tc_tiling_on_sc=False),
        scratch_types=[
            pltpu.VMEM((2 * _B,), jnp.float32),
            pltpu.VMEM((_G * _G,), jnp.float32),
            pltpu.VMEM((_B,), jnp.float32),
            pltpu.SemaphoreType.DMA,
        ],
    )(_tec_body)
    obs_flat = obs.reshape(_N // _G, _G, 2).transpose(0, 2, 1).reshape(-1)
    return run(obs_flat, pdf_grid.reshape(-1)).reshape(_N, 1)


# speedup vs baseline: 1069.1627x; 1.1885x over previous
import functools

import jax
import jax.numpy as jnp
import numpy as np
from jax import lax
from jax.experimental import pallas as pl
from jax.experimental.pallas import tpu as pltpu
from jax.experimental.pallas import tpu_sc as plsc

_N = 1048576
_G = 128
_NC = 2
_NS = 16
_NW = _NC * _NS
_B = _N // _NW
_L = 16
_EPS = np.float32(1e-10)
_SCALE = np.float32(_G - 1)
_GM2 = np.int32(_G - 2)


def _tec_body(obs_hbm, grid_hbm, out_hbm, obs_v, grid_v, out_v, sem):
    wid = lax.axis_index("s") * _NC + lax.axis_index("c")
    base = wid * _B
    c0 = pltpu.async_copy(obs_hbm.at[pl.ds(2 * base, 2 * _B)], obs_v, sem)
    c1 = pltpu.async_copy(grid_hbm, grid_v, sem)
    c0.wait()
    c1.wait()

    @plsc.parallel_loop(0, _B, _L, unroll=8)
    def _loop(o):
        addr = o + ((o >> 7) << 7)
        u = obs_v[pl.ds(addr, _L)]
        v = obs_v[pl.ds(addr + 128, _L)]

        fu = jnp.maximum(u, _EPS) * _SCALE
        fv = jnp.maximum(v, _EPS) * _SCALE
        i0u = jnp.minimum(fu.astype(jnp.int32), _GM2)
        i0v = jnp.minimum(fv.astype(jnp.int32), _GM2)
        du = fu - i0u.astype(jnp.float32)
        dv = fv - i0v.astype(jnp.float32)

        a00 = (i0u << 7) + i0v
        a10 = a00 + _G
        g00 = plsc.load_gather(grid_v, [a00])
        g01 = plsc.load_gather(grid_v, [a00 + 1])
        g10 = plsc.load_gather(grid_v, [a10])
        g11 = plsc.load_gather(grid_v, [a10 + 1])

        top = g00 + (g01 - g00) * dv
        bot = g10 + (g11 - g10) * dv
        res = top + (bot - top) * du
        out_v[pl.ds(o, _L)] = jnp.maximum(res, np.float32(0.0))

    pltpu.sync_copy(out_v, out_hbm.at[0, pl.ds(base, _B)])


def kernel(obs, pdf_grid):
    mesh = plsc.VectorSubcoreMesh(core_axis_name="c", subcore_axis_name="s")
    run = functools.partial(
        pl.kernel,
        mesh=mesh,
        out_type=jax.ShapeDtypeStruct((1, _N), jnp.float32),
        compiler_params=pltpu.CompilerParams(
            needs_layout_passes=False, use_tc_tiling_on_sc=False),
        scratch_types=[
            pltpu.VMEM((2 * _B,), jnp.float32),
            pltpu.VMEM((_G * _G,), jnp.float32),
            pltpu.VMEM((_B,), jnp.float32),
            pltpu.SemaphoreType.DMA,
        ],
    )(_tec_body)
    obs_flat = obs.reshape(_N // _G, _G, 2).transpose(0, 2, 1).reshape(-1)
    return run(obs_flat, pdf_grid.reshape(-1)).reshape(_N, 1)
